# Initial kernel scaffold; baseline (speedup 1.0000x reference)
import jax
import jax.numpy as jnp
from jax.experimental import pallas as pl


def kernel(x, edge_index, W, b):
    raise NotImplementedError("write your pallas kernel here")


# speedup vs baseline: 7.5709x; 7.5709x over previous
import jax
import jax.numpy as jnp
from jax import lax
from jax.experimental import pallas as pl
from jax.experimental.pallas import tpu as pltpu
from jax.experimental.pallas import tpu_sc as plsc

N = 10000
D = 128
NC = 2
NS = 16
NTILE = NC * NS
NPAD = 10240
RPT = NPAD // NS
DUMMY = N
C = 64
EPT = 10240
NCHUNK = EPT // C
EPAD = NTILE * EPT
F32 = jnp.float32
I32 = jnp.int32


def _sc_deg_body(row_hbm, col_hbm, degp_hbm, colp_hbm,
                 row_v, col_v, hist, shared, slab, degp_v):
    c = lax.axis_index("c")
    s = lax.axis_index("s")
    pltpu.sync_copy(row_hbm.at[c, s], row_v)
    pltpu.sync_copy(col_hbm.at[c, s], col_v)

    zero16 = jnp.zeros((16,), F32)
    ones16 = jnp.ones((16,), F32)
    dummy16 = jnp.full((16,), DUMMY, I32)

    def zinit(i, carry):
        hist[pl.ds(i * 16, 16)] = zero16
        return carry
    lax.fori_loop(0, NPAD // 16, zinit, 0)

    def ebody(j, carry):
        for k in range(C // 16):
            r = row_v[pl.ds(j * C + k * 16, 16)]
            cc = col_v[j, pl.ds(k * 16, 16)]
            m = r != cc
            plsc.addupdate_scatter(hist, [r], ones16, mask=m)
            col_v[j, pl.ds(k * 16, 16)] = jnp.where(m, cc, dummy16)
        return carry
    lax.fori_loop(0, NCHUNK, ebody, 0)

    pltpu.sync_copy(col_v, colp_hbm.at[c, s])

    pltpu.sync_copy(hist, shared.at[s])
    plsc.subcore_barrier()
    for t in range(NS):
        pltpu.sync_copy(shared.at[t, pl.ds(s * RPT, RPT)], slab.at[t])

    def rbody(i, carry):
        a = slab[0, pl.ds(i * 16, 16)]
        for t in range(1, NS):
            a = a + slab[t, pl.ds(i * 16, 16)]
        degp_v[pl.ds(i * 16, 16)] = a
        return carry
    lax.fori_loop(0, RPT // 16, rbody, 0)
    pltpu.sync_copy(degp_v, degp_hbm.at[c, pl.ds(s * RPT, RPT)])


def _make_sc_deg(mesh):
    return pl.kernel(
        _sc_deg_body,
        out_type=(jax.ShapeDtypeStruct((NC, NPAD), F32),
                  jax.ShapeDtypeStruct((NC, NS, NCHUNK, C), I32)),
        mesh=mesh,
        compiler_params=pltpu.CompilerParams(needs_layout_passes=False),
        scratch_types=[
            pltpu.VMEM((EPT,), I32),
            pltpu.VMEM((NCHUNK, C), I32),
            pltpu.VMEM((NPAD,), F32),
            pltpu.VMEM_SHARED((NS, NPAD), F32),
            pltpu.VMEM((NS, RPT), F32),
            pltpu.VMEM((RPT,), F32),
        ],
    )


def _sc_prop_body(tab_hbm, row_hbm, colp_hbm, r_hbm,
                  row_v, colp_v, buf0, buf1, acc, gsem0, gsem1):
    c = lax.axis_index("c")
    s = lax.axis_index("s")
    pltpu.sync_copy(row_hbm.at[c, s], row_v)
    pltpu.sync_copy(colp_hbm.at[c, s], colp_v)

    zero16 = jnp.zeros((16,), F32)

    def zb(i, carry):
        for k in range(D // 16):
            buf0[i, pl.ds(k * 16, 16)] = zero16
        return carry
    lax.fori_loop(0, C, zb, 0)
    for i in range(RPT // C):
        pltpu.sync_copy(buf0, acc.at[pl.ds(s * RPT + i * C, C)])
    plsc.subcore_barrier()

    bufs = (buf0, buf1)
    sems = (gsem0, gsem1)
    pltpu.async_copy(tab_hbm.at[row_v.at[pl.ds(0, C)]], buf0, gsem0)
    pltpu.async_copy(tab_hbm.at[row_v.at[pl.ds(C, C)]], buf1, gsem1)

    def chunk(g, carry):
        for b in range(2):
            j = g * 2 + b
            pltpu.make_async_copy(
                tab_hbm.at[row_v.at[pl.ds(j * C, C)]], bufs[b], sems[b]).wait()
            pltpu.sync_copy(bufs[b], acc.at[colp_v.at[j]], add=True)

            @pl.when(j + 2 < NCHUNK)
            def _():
                pltpu.async_copy(
                    tab_hbm.at[row_v.at[pl.ds((j + 2) * C, C)]], bufs[b], sems[b])
        return carry
    lax.fori_loop(0, NCHUNK // 2, chunk, 0)

    plsc.subcore_barrier()
    pltpu.sync_copy(acc.at[pl.ds(s * RPT, RPT)],
                    r_hbm.at[c, pl.ds(s * RPT, RPT)])


def _make_sc_prop(mesh):
    return pl.kernel(
        _sc_prop_body,
        out_type=jax.ShapeDtypeStruct((NC, NPAD, D), F32),
        mesh=mesh,
        compiler_params=pltpu.CompilerParams(needs_layout_passes=False),
        scratch_types=[
            pltpu.VMEM((EPT,), I32),
            pltpu.VMEM((NCHUNK, C), I32),
            pltpu.VMEM((C, D), F32),
            pltpu.VMEM((C, D), F32),
            pltpu.VMEM_SHARED((NPAD, D), F32),
            pltpu.SemaphoreType.DMA,
            pltpu.SemaphoreType.DMA,
        ],
    )


def _tc_a_body(degT_ref, x_ref, dis_ref, s_ref):
    dsum = degT_ref[:, 0:1] + degT_ref[:, 1:2]
    pos = dsum > 0.0
    dis = jnp.where(pos, lax.rsqrt(jnp.where(pos, dsum, 1.0)), 0.0)
    dis_ref[...] = dis
    s_ref[...] = dis * x_ref[...]


_tc_a = pl.pallas_call(
    _tc_a_body,
    out_shape=(jax.ShapeDtypeStruct((N, 1), F32),
               jax.ShapeDtypeStruct((N, D), F32)),
)


def _tc_b_body(r1a_ref, r1b_ref, dis_ref, tx1_ref, s2_ref):
    dis = dis_ref[...]
    t = (r1a_ref[...] + r1b_ref[...]) * (-dis)
    tx1_ref[...] = t
    s2_ref[...] = dis * t


_tc_b = pl.pallas_call(
    _tc_b_body,
    out_shape=(jax.ShapeDtypeStruct((N, D), F32),
               jax.ShapeDtypeStruct((N, D), F32)),
)


def _tc_c_body(x_ref, tx1_ref, r2a_ref, r2b_ref, dis_ref, w_ref, b_ref, out_ref):
    x = x_ref[...]
    tx2 = (r2a_ref[...] + r2b_ref[...]) * (-2.0 * dis_ref[...]) - x
    out = jnp.dot(x, w_ref[0], preferred_element_type=F32)
    out = out + jnp.dot(tx1_ref[...], w_ref[1], preferred_element_type=F32)
    out = out + jnp.dot(tx2, w_ref[2], preferred_element_type=F32)
    out_ref[...] = out + b_ref[...]


_tc_c = pl.pallas_call(
    _tc_c_body,
    out_shape=jax.ShapeDtypeStruct((N, D), F32),
)


def kernel(x, edge_index, W, b):
    row = edge_index[0].astype(I32)
    col = edge_index[1].astype(I32)
    e = row.shape[0]
    pad = EPAD - e
    row_t = jnp.concatenate([row, jnp.zeros((pad,), I32)]).reshape(NC, NS, EPT)
    col_t = jnp.concatenate([col, jnp.zeros((pad,), I32)]).reshape(NC, NS, NCHUNK, C)

    mesh = plsc.VectorSubcoreMesh(core_axis_name="c", subcore_axis_name="s")
    degp, colp_t = _make_sc_deg(mesh)(row_t, col_t)
    degT = degp[:, :N].T
    dis, s = _tc_a(degT, x)

    prop = _make_sc_prop(mesh)
    r1 = prop(s, row_t, colp_t)
    tx1, s2 = _tc_b(r1[0, :N], r1[1, :N], dis)
    r2 = prop(s2, row_t, colp_t)
    out = _tc_c(x, tx1, r2[0, :N], r2[1, :N], dis, W, b.reshape(1, D))
    return out


# speedup vs baseline: 8.1447x; 1.0758x over previous
import jax
import jax.numpy as jnp
from jax import lax
from jax.experimental import pallas as pl
from jax.experimental.pallas import tpu as pltpu
from jax.experimental.pallas import tpu_sc as plsc

N = 10000
D = 128
NC = 2
NS = 16
NTILE = NC * NS
NPAD = 10240
RPT = NPAD // NS
DUMMY = N
C = 64
EPT = 10240
NCHUNK = EPT // C
EPAD = NTILE * EPT
F32 = jnp.float32
I32 = jnp.int32


def _sc_deg_body(row_hbm, col_hbm, degp_hbm, colp_hbm,
                 row_v, col_v, hist, shared, slab, degp_v):
    c = lax.axis_index("c")
    s = lax.axis_index("s")
    pltpu.sync_copy(row_hbm.at[c, s], row_v)
    pltpu.sync_copy(col_hbm.at[c, s], col_v)

    zero16 = jnp.zeros((16,), F32)
    ones16 = jnp.ones((16,), F32)
    n16 = jnp.full((16,), N, I32)
    dummy16 = N + lax.iota(I32, 16)

    def zinit(i, carry):
        hist[pl.ds(i * 16, 16)] = zero16
        return carry
    lax.fori_loop(0, NPAD // 16, zinit, 0)

    def ebody(j, carry):
        for k in range(C // 16):
            r = row_v[pl.ds(j * C + k * 16, 16)]
            cc = col_v[j, pl.ds(k * 16, 16)]
            keep = r != cc
            m = keep & (cc < n16)
            plsc.addupdate_scatter(hist, [r], ones16, mask=m)
            col_v[j, pl.ds(k * 16, 16)] = jnp.where(keep, cc, dummy16)
        return carry
    lax.fori_loop(0, NCHUNK, ebody, 0)

    pltpu.sync_copy(col_v, colp_hbm.at[c, s])

    pltpu.sync_copy(hist, shared.at[s])
    plsc.subcore_barrier()
    for t in range(NS):
        pltpu.sync_copy(shared.at[t, pl.ds(s * RPT, RPT)], slab.at[t])

    def rbody(i, carry):
        a = slab[0, pl.ds(i * 16, 16)]
        for t in range(1, NS):
            a = a + slab[t, pl.ds(i * 16, 16)]
        degp_v[pl.ds(i * 16, 16)] = a
        return carry
    lax.fori_loop(0, RPT // 16, rbody, 0)
    pltpu.sync_copy(degp_v, degp_hbm.at[c, pl.ds(s * RPT, RPT)])


def _make_sc_deg(mesh):
    return pl.kernel(
        _sc_deg_body,
        out_type=(jax.ShapeDtypeStruct((NC, NPAD), F32),
                  jax.ShapeDtypeStruct((NC, NS, NCHUNK, C), I32)),
        mesh=mesh,
        compiler_params=pltpu.CompilerParams(needs_layout_passes=False),
        scratch_types=[
            pltpu.VMEM((EPT,), I32),
            pltpu.VMEM((NCHUNK, C), I32),
            pltpu.VMEM((NPAD,), F32),
            pltpu.VMEM_SHARED((NS, NPAD), F32),
            pltpu.VMEM((NS, RPT), F32),
            pltpu.VMEM((RPT,), F32),
        ],
    )


def _sc_prop_body(tab_hbm, row_hbm, colp_hbm, r_hbm,
                  row_v, colp_v, buf0, buf1, acc, gsem0, gsem1):
    c = lax.axis_index("c")
    s = lax.axis_index("s")
    pltpu.sync_copy(row_hbm.at[c, s], row_v)
    pltpu.sync_copy(colp_hbm.at[c, s], colp_v)

    zero16 = jnp.zeros((16,), F32)

    def zb(i, carry):
        for k in range(D // 16):
            buf0[i, pl.ds(k * 16, 16)] = zero16
        return carry
    lax.fori_loop(0, C, zb, 0)
    for i in range(RPT // C):
        pltpu.sync_copy(buf0, acc.at[pl.ds(s * RPT + i * C, C)])
    plsc.subcore_barrier()

    bufs = (buf0, buf1)
    sems = (gsem0, gsem1)
    pltpu.async_copy(tab_hbm.at[row_v.at[pl.ds(0, C)]], buf0, gsem0)
    pltpu.async_copy(tab_hbm.at[row_v.at[pl.ds(C, C)]], buf1, gsem1)

    def chunk(g, carry):
        for b in range(2):
            j = g * 2 + b
            pltpu.make_async_copy(
                tab_hbm.at[row_v.at[pl.ds(j * C, C)]], bufs[b], sems[b]).wait()
            pltpu.sync_copy(bufs[b], acc.at[colp_v.at[j]], add=True)

            @pl.when(j + 2 < NCHUNK)
            def _():
                pltpu.async_copy(
                    tab_hbm.at[row_v.at[pl.ds((j + 2) * C, C)]], bufs[b], sems[b])
        return carry
    lax.fori_loop(0, NCHUNK // 2, chunk, 0)

    plsc.subcore_barrier()
    pltpu.sync_copy(acc.at[pl.ds(s * RPT, RPT)],
                    r_hbm.at[c, pl.ds(s * RPT, RPT)])


def _make_sc_prop(mesh):
    return pl.kernel(
        _sc_prop_body,
        out_type=jax.ShapeDtypeStruct((NC, NPAD, D), F32),
        mesh=mesh,
        compiler_params=pltpu.CompilerParams(needs_layout_passes=False),
        scratch_types=[
            pltpu.VMEM((EPT,), I32),
            pltpu.VMEM((NCHUNK, C), I32),
            pltpu.VMEM((C, D), F32),
            pltpu.VMEM((C, D), F32),
            pltpu.VMEM_SHARED((NPAD, D), F32),
            pltpu.SemaphoreType.DMA,
            pltpu.SemaphoreType.DMA,
        ],
    )


def _tc_a_body(degT_ref, x_ref, dis_ref, s_ref):
    dsum = degT_ref[:, 0:1] + degT_ref[:, 1:2]
    pos = dsum > 0.0
    dis = jnp.where(pos, lax.rsqrt(jnp.where(pos, dsum, 1.0)), 0.0)
    dis_ref[...] = dis
    s_ref[...] = dis * x_ref[...]


_tc_a = pl.pallas_call(
    _tc_a_body,
    out_shape=(jax.ShapeDtypeStruct((N, 1), F32),
               jax.ShapeDtypeStruct((N, D), F32)),
)


def _tc_b_body(r1a_ref, r1b_ref, dis_ref, tx1_ref, s2_ref):
    dis = dis_ref[...]
    t = (r1a_ref[...] + r1b_ref[...]) * (-dis)
    tx1_ref[...] = t
    s2_ref[...] = dis * t


_tc_b = pl.pallas_call(
    _tc_b_body,
    out_shape=(jax.ShapeDtypeStruct((N, D), F32),
               jax.ShapeDtypeStruct((N, D), F32)),
)


def _tc_c_body(x_ref, tx1_ref, r2a_ref, r2b_ref, dis_ref, w_ref, b_ref, out_ref):
    x = x_ref[...]
    tx2 = (r2a_ref[...] + r2b_ref[...]) * (-2.0 * dis_ref[...]) - x
    out = jnp.dot(x, w_ref[0], preferred_element_type=F32)
    out = out + jnp.dot(tx1_ref[...], w_ref[1], preferred_element_type=F32)
    out = out + jnp.dot(tx2, w_ref[2], preferred_element_type=F32)
    out_ref[...] = out + b_ref[...]


_tc_c = pl.pallas_call(
    _tc_c_body,
    out_shape=jax.ShapeDtypeStruct((N, D), F32),
)


def kernel(x, edge_index, W, b):
    row = edge_index[0].astype(I32)
    col = edge_index[1].astype(I32)
    e = row.shape[0]
    ept_real = e // NTILE
    ppt = EPT - ept_real
    pad_col = (N + jnp.arange(NTILE * ppt, dtype=I32) % (NPAD - N)).reshape(NTILE, ppt)
    row_t = jnp.concatenate(
        [row.reshape(NTILE, ept_real), jnp.zeros((NTILE, ppt), I32)],
        axis=1).reshape(NC, NS, EPT)
    col_t = jnp.concatenate(
        [col.reshape(NTILE, ept_real), pad_col],
        axis=1).reshape(NC, NS, NCHUNK, C)

    mesh = plsc.VectorSubcoreMesh(core_axis_name="c", subcore_axis_name="s")
    degp, colp_t = _make_sc_deg(mesh)(row_t, col_t)
    degT = degp[:, :N].T
    dis, s = _tc_a(degT, x)

    prop = _make_sc_prop(mesh)
    r1 = prop(s, row_t, colp_t)
    tx1, s2 = _tc_b(r1[0, :N], r1[1, :N], dis)
    r2 = prop(s2, row_t, colp_t)
    out = _tc_c(x, tx1, r2[0, :N], r2[1, :N], dis, W, b.reshape(1, D))
    return out


# speedup vs baseline: 15.9963x; 1.9640x over previous
import jax
import jax.numpy as jnp
from jax import lax
from jax.experimental import pallas as pl
from jax.experimental.pallas import tpu as pltpu
from jax.experimental.pallas import tpu_sc as plsc

N = 10000
D = 128
DH = D // 2
NC = 2
NS = 16
NTILE = NC * NS
NPAD = 10240
RPT = NPAD // NS
SPT = N // NS
C = 64
EPT = 10240
NCHUNK = EPT // C
EPAD = NTILE * EPT
F32 = jnp.float32
I32 = jnp.int32

_SC_PARAMS = pltpu.CompilerParams(needs_layout_passes=False,
                                  use_tc_tiling_on_sc=False)


def _sc_deg_body(row_hbm, col_hbm, degp_hbm, colp_hbm,
                 row_v, col_v, hist, shared, slab, degp_v):
    c = lax.axis_index("c")
    s = lax.axis_index("s")
    pltpu.sync_copy(row_hbm.at[c, s], row_v)
    pltpu.sync_copy(col_hbm.at[c, s], col_v)

    zero16 = jnp.zeros((16,), F32)
    ones16 = jnp.ones((16,), F32)
    n16 = jnp.full((16,), N, I32)
    dummy16 = N + lax.iota(I32, 16)

    def zinit(i, carry):
        hist[pl.ds(i * 16, 16)] = zero16
        return carry
    lax.fori_loop(0, NPAD // 16, zinit, 0)

    def ebody(j, carry):
        for k in range(C // 16):
            r = row_v[pl.ds(j * C + k * 16, 16)]
            cc = col_v[j, pl.ds(k * 16, 16)]
            keep = r != cc
            m = keep & (cc < n16)
            plsc.addupdate_scatter(hist, [r], ones16, mask=m)
            col_v[j, pl.ds(k * 16, 16)] = jnp.where(keep, cc, dummy16)
        return carry
    lax.fori_loop(0, NCHUNK, ebody, 0)

    pltpu.sync_copy(col_v, colp_hbm.at[c, s])

    pltpu.sync_copy(hist, shared.at[s])
    plsc.subcore_barrier()
    for t in range(NS):
        pltpu.sync_copy(shared.at[t, pl.ds(s * RPT, RPT)], slab.at[t])

    def rbody(i, carry):
        a = slab[0, pl.ds(i * 16, 16)]
        for t in range(1, NS):
            a = a + slab[t, pl.ds(i * 16, 16)]
        degp_v[pl.ds(i * 16, 16)] = a
        return carry
    lax.fori_loop(0, RPT // 16, rbody, 0)
    pltpu.sync_copy(degp_v, degp_hbm.at[c, pl.ds(s * RPT, RPT)])


def _make_sc_deg(mesh):
    return pl.kernel(
        _sc_deg_body,
        out_type=(jax.ShapeDtypeStruct((NC, NPAD), F32),
                  jax.ShapeDtypeStruct((NC, NS, NCHUNK, C), I32)),
        mesh=mesh,
        compiler_params=_SC_PARAMS,
        scratch_types=[
            pltpu.VMEM((EPT,), I32),
            pltpu.VMEM((NCHUNK, C), I32),
            pltpu.VMEM((NPAD,), F32),
            pltpu.VMEM_SHARED((NS, NPAD), F32),
            pltpu.VMEM((NS, RPT), F32),
            pltpu.VMEM((RPT,), F32),
        ],
    )


def _sc_prop_body(slo_hbm, shi_hbm, row_hbm, colp_hbm, r_hbm,
                  row_v, colp_v, buf0, buf1, table, acc, gsem0, gsem1):
    c = lax.axis_index("c")
    s = lax.axis_index("s")
    pltpu.sync_copy(row_hbm.at[c, s], row_v)
    pltpu.sync_copy(colp_hbm.at[c, s], colp_v)

    zero16 = jnp.zeros((16,), F32)

    def zb(i, carry):
        for k in range(DH // 16):
            buf0[i, pl.ds(k * 16, 16)] = zero16
        return carry
    lax.fori_loop(0, C, zb, 0)

    bufs = (buf0, buf1)
    sems = (gsem0, gsem1)
    for p, s_hbm in enumerate((slo_hbm, shi_hbm)):
        pltpu.sync_copy(s_hbm.at[pl.ds(s * SPT, SPT)],
                        table.at[pl.ds(s * SPT, SPT)])
        for i in range(RPT // C):
            pltpu.sync_copy(buf0, acc.at[pl.ds(s * RPT + i * C, C)])
        plsc.subcore_barrier()

        pltpu.async_copy(table.at[row_v.at[pl.ds(0, C)]], buf0, gsem0)
        pltpu.async_copy(table.at[row_v.at[pl.ds(C, C)]], buf1, gsem1)

        def chunk(g, carry):
            for b in range(2):
                j = g * 2 + b
                pltpu.make_async_copy(
                    table.at[row_v.at[pl.ds(j * C, C)]], bufs[b], sems[b]).wait()
                pltpu.sync_copy(bufs[b], acc.at[colp_v.at[j]], add=True)

                @pl.when(j + 2 < NCHUNK)
                def _():
                    pltpu.async_copy(
                        table.at[row_v.at[pl.ds((j + 2) * C, C)]], bufs[b], sems[b])
            return carry
        lax.fori_loop(0, NCHUNK // 2, chunk, 0)

        plsc.subcore_barrier()
        pltpu.sync_copy(acc.at[pl.ds(s * RPT, RPT)],
                        r_hbm.at[c, p, pl.ds(s * RPT, RPT)])
        if p == 0:
            def rz(i, carry):
                for k in range(DH // 16):
                    buf0[i, pl.ds(k * 16, 16)] = zero16
                return carry
            lax.fori_loop(0, C, rz, 0)
            plsc.subcore_barrier()


def _make_sc_prop(mesh):
    return pl.kernel(
        _sc_prop_body,
        out_type=jax.ShapeDtypeStruct((NC, 2, NPAD, DH), F32),
        mesh=mesh,
        compiler_params=_SC_PARAMS,
        scratch_types=[
            pltpu.VMEM((EPT,), I32),
            pltpu.VMEM((NCHUNK, C), I32),
            pltpu.VMEM((C, DH), F32),
            pltpu.VMEM((C, DH), F32),
            pltpu.VMEM_SHARED((N, DH), F32),
            pltpu.VMEM_SHARED((NPAD, DH), F32),
            pltpu.SemaphoreType.DMA,
            pltpu.SemaphoreType.DMA,
        ],
    )


def _tc_a_body(degT_ref, x_ref, dis_ref, slo_ref, shi_ref):
    dsum = degT_ref[:, 0:1] + degT_ref[:, 1:2]
    pos = dsum > 0.0
    dis = jnp.where(pos, lax.rsqrt(jnp.where(pos, dsum, 1.0)), 0.0)
    dis_ref[...] = dis
    sx = dis * x_ref[...]
    slo_ref[...] = sx[:, :DH]
    shi_ref[...] = sx[:, DH:]


_tc_a = pl.pallas_call(
    _tc_a_body,
    out_shape=(jax.ShapeDtypeStruct((N, 1), F32),
               jax.ShapeDtypeStruct((N, DH), F32),
               jax.ShapeDtypeStruct((N, DH), F32)),
)


def _tc_b_body(r1al_ref, r1ah_ref, r1bl_ref, r1bh_ref, dis_ref,
               tx1_ref, s2lo_ref, s2hi_ref):
    dis = dis_ref[...]
    tlo = (r1al_ref[...] + r1bl_ref[...]) * (-dis)
    thi = (r1ah_ref[...] + r1bh_ref[...]) * (-dis)
    tx1_ref[...] = jnp.concatenate([tlo, thi], axis=1)
    s2lo_ref[...] = dis * tlo
    s2hi_ref[...] = dis * thi


_tc_b = pl.pallas_call(
    _tc_b_body,
    out_shape=(jax.ShapeDtypeStruct((N, D), F32),
               jax.ShapeDtypeStruct((N, DH), F32),
               jax.ShapeDtypeStruct((N, DH), F32)),
)


def _tc_c_body(x_ref, tx1_ref, r2al_ref, r2ah_ref, r2bl_ref, r2bh_ref,
               dis_ref, w_ref, b_ref, out_ref):
    x = x_ref[...]
    m2dis = -2.0 * dis_ref[...]
    tx2 = jnp.concatenate(
        [(r2al_ref[...] + r2bl_ref[...]) * m2dis,
         (r2ah_ref[...] + r2bh_ref[...]) * m2dis], axis=1) - x
    out = jnp.dot(x, w_ref[0], preferred_element_type=F32)
    out = out + jnp.dot(tx1_ref[...], w_ref[1], preferred_element_type=F32)
    out = out + jnp.dot(tx2, w_ref[2], preferred_element_type=F32)
    out_ref[...] = out + b_ref[...]


_tc_c = pl.pallas_call(
    _tc_c_body,
    out_shape=jax.ShapeDtypeStruct((N, D), F32),
)


def kernel(x, edge_index, W, b):
    row = edge_index[0].astype(I32)
    col = edge_index[1].astype(I32)
    e = row.shape[0]
    ept_real = e // NTILE
    ppt = EPT - ept_real
    pad_col = (N + jnp.arange(NTILE * ppt, dtype=I32) % (NPAD - N)).reshape(NTILE, ppt)
    row_t = jnp.concatenate(
        [row.reshape(NTILE, ept_real), jnp.zeros((NTILE, ppt), I32)],
        axis=1).reshape(NC, NS, EPT)
    col_t = jnp.concatenate(
        [col.reshape(NTILE, ept_real), pad_col],
        axis=1).reshape(NC, NS, NCHUNK, C)

    mesh = plsc.VectorSubcoreMesh(core_axis_name="c", subcore_axis_name="s")
    degp, colp_t = _make_sc_deg(mesh)(row_t, col_t)
    degT = degp[:, :N].T
    dis, slo, shi = _tc_a(degT, x)

    prop = _make_sc_prop(mesh)
    r1 = prop(slo, shi, row_t, colp_t)
    tx1, s2lo, s2hi = _tc_b(r1[0, 0, :N], r1[0, 1, :N],
                            r1[1, 0, :N], r1[1, 1, :N], dis)
    r2 = prop(s2lo, s2hi, row_t, colp_t)
    out = _tc_c(x, tx1, r2[0, 0, :N], r2[0, 1, :N],
                r2[1, 0, :N], r2[1, 1, :N], dis, W, b.reshape(1, D))
    return out


# speedup vs baseline: 17.7281x; 1.1083x over previous
import jax
import jax.numpy as jnp
from jax import lax
from jax.experimental import pallas as pl
from jax.experimental.pallas import tpu as pltpu
from jax.experimental.pallas import tpu_sc as plsc

N = 10000
D = 128
DH = D // 2
NC = 2
NS = 16
NTILE = NC * NS
NPAD = 10240
RPT = NPAD // NS
SPT = N // NS
C = 64
EPT = 10240
NCHUNK = EPT // C
EPAD = NTILE * EPT
F32 = jnp.float32
I32 = jnp.int32

_SC_PARAMS = pltpu.CompilerParams(needs_layout_passes=False,
                                  use_tc_tiling_on_sc=False)


def _sc_deg_body(row_hbm, col_hbm, degp_hbm, colp_hbm,
                 row_v, col_v, hist, shared, slab, degp_v):
    c = lax.axis_index("c")
    s = lax.axis_index("s")
    pltpu.sync_copy(row_hbm.at[c, s], row_v)
    pltpu.sync_copy(col_hbm.at[c, s], col_v)

    zero16 = jnp.zeros((16,), F32)
    ones16 = jnp.ones((16,), F32)
    n16 = jnp.full((16,), N, I32)
    dummy16 = N + lax.iota(I32, 16)

    def zinit(i, carry):
        hist[pl.ds(i * 16, 16)] = zero16
        return carry
    lax.fori_loop(0, NPAD // 16, zinit, 0)

    def ebody(j, carry):
        for k in range(C // 16):
            r = row_v[pl.ds(j * C + k * 16, 16)]
            cc = col_v[j, pl.ds(k * 16, 16)]
            keep = r != cc
            m = keep & (cc < n16)
            plsc.addupdate_scatter(hist, [r], ones16, mask=m)
            col_v[j, pl.ds(k * 16, 16)] = jnp.where(keep, cc, dummy16)
        return carry
    lax.fori_loop(0, NCHUNK, ebody, 0)

    pltpu.sync_copy(col_v, colp_hbm.at[c, s])

    pltpu.sync_copy(hist, shared.at[s])
    plsc.subcore_barrier()
    for t in range(NS):
        pltpu.sync_copy(shared.at[t, pl.ds(s * RPT, RPT)], slab.at[t])

    def rbody(i, carry):
        a = slab[0, pl.ds(i * 16, 16)]
        for t in range(1, NS):
            a = a + slab[t, pl.ds(i * 16, 16)]
        degp_v[pl.ds(i * 16, 16)] = a
        return carry
    lax.fori_loop(0, RPT // 16, rbody, 0)
    pltpu.sync_copy(degp_v, degp_hbm.at[c, pl.ds(s * RPT, RPT)])


def _make_sc_deg(mesh):
    return pl.kernel(
        _sc_deg_body,
        out_type=(jax.ShapeDtypeStruct((NC, NPAD), F32),
                  jax.ShapeDtypeStruct((NC, NS, NCHUNK, C), I32)),
        mesh=mesh,
        compiler_params=_SC_PARAMS,
        scratch_types=[
            pltpu.VMEM((EPT,), I32),
            pltpu.VMEM((NCHUNK, C), I32),
            pltpu.VMEM((NPAD,), F32),
            pltpu.VMEM_SHARED((NS, NPAD), F32),
            pltpu.VMEM((NS, RPT), F32),
            pltpu.VMEM((RPT,), F32),
        ],
    )


NB = 6
LOOKA = NB // 2


def _sc_prop_body(slo_hbm, shi_hbm, row_hbm, colp_hbm, r_hbm,
                  row_v, colp_v, *rest):
    bufs = rest[:NB]
    table, acc = rest[NB], rest[NB + 1]
    gsems = rest[NB + 2:NB + 2 + NB]
    ssems = rest[NB + 2 + NB:]
    c = lax.axis_index("c")
    s = lax.axis_index("s")
    pltpu.sync_copy(row_hbm.at[c, s], row_v)
    pltpu.sync_copy(colp_hbm.at[c, s], colp_v)

    zero16 = jnp.zeros((16,), F32)

    def zb(i, carry):
        for k in range(DH // 16):
            bufs[0][i, pl.ds(k * 16, 16)] = zero16
        return carry
    lax.fori_loop(0, C, zb, 0)

    nround = (NCHUNK + LOOKA + NB) // NB + 1
    for p, s_hbm in enumerate((slo_hbm, shi_hbm)):
        pltpu.sync_copy(s_hbm.at[pl.ds(s * SPT, SPT)],
                        table.at[pl.ds(s * SPT, SPT)])
        for i in range(RPT // C):
            pltpu.sync_copy(bufs[0], acc.at[pl.ds(s * RPT + i * C, C)])
        plsc.subcore_barrier()

        def round_(g, carry):
            for b in range(NB):
                k = g * NB + b

                @pl.when((k >= NB) & (k < NCHUNK + NB))
                def _():
                    pltpu.make_async_copy(
                        bufs[b], acc.at[colp_v.at[k - NB]], ssems[b]).wait()

                @pl.when(k < NCHUNK)
                def _():
                    pltpu.async_copy(
                        table.at[row_v.at[pl.ds(k * C, C)]], bufs[b], gsems[b])

                j = k - LOOKA
                bj = (b - LOOKA) % NB

                @pl.when((j >= 0) & (j < NCHUNK))
                def _():
                    pltpu.make_async_copy(
                        table.at[row_v.at[pl.ds(j * C, C)]], bufs[bj],
                        gsems[bj]).wait()
                    pltpu.async_copy(bufs[bj], acc.at[colp_v.at[j]],
                                     ssems[bj], add=True)
            return carry
        lax.fori_loop(0, nround, round_, 0)

        plsc.subcore_barrier()
        pltpu.sync_copy(acc.at[pl.ds(s * RPT, RPT)],
                        r_hbm.at[c, p, pl.ds(s * RPT, RPT)])
        if p == 0:
            def rz(i, carry):
                for k in range(DH // 16):
                    bufs[0][i, pl.ds(k * 16, 16)] = zero16
                return carry
            lax.fori_loop(0, C, rz, 0)
            plsc.subcore_barrier()


def _make_sc_prop(mesh):
    return pl.kernel(
        _sc_prop_body,
        out_type=jax.ShapeDtypeStruct((NC, 2, NPAD, DH), F32),
        mesh=mesh,
        compiler_params=_SC_PARAMS,
        scratch_types=(
            [pltpu.VMEM((EPT,), I32),
             pltpu.VMEM((NCHUNK, C), I32)]
            + [pltpu.VMEM((C, DH), F32) for _ in range(NB)]
            + [pltpu.VMEM_SHARED((N, DH), F32),
               pltpu.VMEM_SHARED((NPAD, DH), F32)]
            + [pltpu.SemaphoreType.DMA for _ in range(2 * NB)]
        ),
    )


def _tc_a_body(degT_ref, x_ref, dis_ref, slo_ref, shi_ref):
    dsum = degT_ref[:, 0:1] + degT_ref[:, 1:2]
    pos = dsum > 0.0
    dis = jnp.where(pos, lax.rsqrt(jnp.where(pos, dsum, 1.0)), 0.0)
    dis_ref[...] = dis
    sx = dis * x_ref[...]
    slo_ref[...] = sx[:, :DH]
    shi_ref[...] = sx[:, DH:]


_tc_a = pl.pallas_call(
    _tc_a_body,
    out_shape=(jax.ShapeDtypeStruct((N, 1), F32),
               jax.ShapeDtypeStruct((N, DH), F32),
               jax.ShapeDtypeStruct((N, DH), F32)),
)


def _tc_b_body(r1al_ref, r1ah_ref, r1bl_ref, r1bh_ref, dis_ref,
               tx1_ref, s2lo_ref, s2hi_ref):
    dis = dis_ref[...]
    tlo = (r1al_ref[...] + r1bl_ref[...]) * (-dis)
    thi = (r1ah_ref[...] + r1bh_ref[...]) * (-dis)
    tx1_ref[...] = jnp.concatenate([tlo, thi], axis=1)
    s2lo_ref[...] = dis * tlo
    s2hi_ref[...] = dis * thi


_tc_b = pl.pallas_call(
    _tc_b_body,
    out_shape=(jax.ShapeDtypeStruct((N, D), F32),
               jax.ShapeDtypeStruct((N, DH), F32),
               jax.ShapeDtypeStruct((N, DH), F32)),
)


def _tc_c_body(x_ref, tx1_ref, r2al_ref, r2ah_ref, r2bl_ref, r2bh_ref,
               dis_ref, w_ref, b_ref, out_ref):
    x = x_ref[...]
    m2dis = -2.0 * dis_ref[...]
    tx2 = jnp.concatenate(
        [(r2al_ref[...] + r2bl_ref[...]) * m2dis,
         (r2ah_ref[...] + r2bh_ref[...]) * m2dis], axis=1) - x
    out = jnp.dot(x, w_ref[0], preferred_element_type=F32)
    out = out + jnp.dot(tx1_ref[...], w_ref[1], preferred_element_type=F32)
    out = out + jnp.dot(tx2, w_ref[2], preferred_element_type=F32)
    out_ref[...] = out + b_ref[...]


_tc_c = pl.pallas_call(
    _tc_c_body,
    out_shape=jax.ShapeDtypeStruct((N, D), F32),
)


def kernel(x, edge_index, W, b):
    row = edge_index[0].astype(I32)
    col = edge_index[1].astype(I32)
    e = row.shape[0]
    ept_real = e // NTILE
    ppt = EPT - ept_real
    pad_col = (N + jnp.arange(NTILE * ppt, dtype=I32) % (NPAD - N)).reshape(NTILE, ppt)
    row_t = jnp.concatenate(
        [row.reshape(NTILE, ept_real), jnp.zeros((NTILE, ppt), I32)],
        axis=1).reshape(NC, NS, EPT)
    col_t = jnp.concatenate(
        [col.reshape(NTILE, ept_real), pad_col],
        axis=1).reshape(NC, NS, NCHUNK, C)

    mesh = plsc.VectorSubcoreMesh(core_axis_name="c", subcore_axis_name="s")
    degp, colp_t = _make_sc_deg(mesh)(row_t, col_t)
    degT = degp[:, :N].T
    dis, slo, shi = _tc_a(degT, x)

    prop = _make_sc_prop(mesh)
    r1 = prop(slo, shi, row_t, colp_t)
    tx1, s2lo, s2hi = _tc_b(r1[0, 0, :N], r1[0, 1, :N],
                            r1[1, 0, :N], r1[1, 1, :N], dis)
    r2 = prop(s2lo, s2hi, row_t, colp_t)
    out = _tc_c(x, tx1, r2[0, 0, :N], r2[0, 1, :N],
                r2[1, 0, :N], r2[1, 1, :N], dis, W, b.reshape(1, D))
    return out
